# baseline (speedup 1.0000x reference)
import functools

import jax
import jax.numpy as jnp
from jax import lax
from jax.experimental import pallas as pl
from jax.experimental.pallas import tpu as pltpu
from jax.experimental.pallas import tpu_sc as plsc

EPS = 1e-5
KNB = 32
PTS = 128
KNN_RT = 512


def _uv_body(x_ref, pc_ref, wa_ref, wb_ref, b_ref, u_ref, v_ref):
    x = x_ref[0]
    p = pc_ref[0]
    v = jnp.dot(p, wb_ref[...], preferred_element_type=jnp.float32, precision=lax.Precision.HIGHEST)
    u = (jnp.dot(x, wa_ref[...], preferred_element_type=jnp.float32, precision=lax.Precision.HIGHEST)
         + v + b_ref[...])
    u_ref[0] = u
    v_ref[0] = v


def _compute_uv(x, pc8, wa, wb, b1):
    B, N, Cin = x.shape
    C = wa.shape[1]
    return pl.pallas_call(
        _uv_body,
        grid=(B,),
        in_specs=[
            pl.BlockSpec((1, N, Cin), lambda b: (b, 0, 0)),
            pl.BlockSpec((1, N, 8), lambda b: (b, 0, 0)),
            pl.BlockSpec((Cin, C), lambda b: (0, 0)),
            pl.BlockSpec((8, C), lambda b: (0, 0)),
            pl.BlockSpec((1, C), lambda b: (0, 0)),
        ],
        out_specs=[
            pl.BlockSpec((1, N, C), lambda b: (b, 0, 0)),
            pl.BlockSpec((1, N, C), lambda b: (b, 0, 0)),
        ],
        out_shape=[
            jax.ShapeDtypeStruct((B, N, C), jnp.float32),
            jax.ShapeDtypeStruct((B, N, C), jnp.float32),
        ],
    )(x, pc8, wa, wb, b1.reshape(1, C))


def _knn_body(q_ref, pt_ref, q8_ref, p8t_ref, idx_ref):
    b = pl.program_id(0)
    q = q_ref[0]
    pt = pt_ref[0]
    n = pt.shape[1]
    rt = q.shape[0]
    e = jnp.dot(q8_ref[0], p8t_ref[0], preferred_element_type=jnp.float32,
                precision=lax.Precision.HIGHEST)
    d = (q[:, 3:4] + pt[3:4, :]) - 2.0 * e
    iota = lax.broadcasted_iota(jnp.int32, (rt, n), 1)
    base = b * n
    cols = []
    for _ in range(KNB):
        m = jnp.min(d, axis=1, keepdims=True)
        eq = d == m
        cols.append(jnp.min(jnp.where(eq, iota, n), axis=1) + base)
        d = jnp.where(eq, jnp.float32(jnp.inf), d)
    idx_ref[0] = jnp.stack(cols, axis=1)


def _knn(pcq, pct, pc8, pc8t):
    B, N, _ = pcq.shape
    return pl.pallas_call(
        _knn_body,
        grid=(B, N // KNN_RT),
        in_specs=[
            pl.BlockSpec((1, KNN_RT, 4), lambda b, t: (b, t, 0)),
            pl.BlockSpec((1, 4, N), lambda b, t: (b, 0, 0)),
            pl.BlockSpec((1, KNN_RT, 8), lambda b, t: (b, t, 0)),
            pl.BlockSpec((1, 8, N), lambda b, t: (b, 0, 0)),
        ],
        out_specs=pl.BlockSpec((1, KNN_RT, KNB), lambda b, t: (b, t, 0)),
        out_shape=jax.ShapeDtypeStruct((B, N, KNB), jnp.int32),
    )(pcq, pct, pc8, pc8t)


def _gather_rows(table, idx_flat):
    R, C = idx_flat.shape[0], table.shape[1]
    NW = 32
    CH = 128
    per = R // NW
    nch = per // CH
    mesh = plsc.VectorSubcoreMesh(core_axis_name="c", subcore_axis_name="s")

    @functools.partial(
        pl.kernel,
        mesh=mesh,
        out_type=jax.ShapeDtypeStruct((R, C), jnp.float32),
        scratch_types=[
            pltpu.VMEM((CH,), jnp.int32),
            pltpu.VMEM((CH,), jnp.int32),
            pltpu.VMEM((CH, C), jnp.float32),
            pltpu.VMEM((CH, C), jnp.float32),
            pltpu.SemaphoreType.DMA,
            pltpu.SemaphoreType.DMA,
        ],
    )
    def gk(table_hbm, idx_hbm, out_hbm, idx_a, idx_b, rows_a, rows_b,
           sem_a, sem_b):
        wid = lax.axis_index("s") * 2 + lax.axis_index("c")
        base = wid * per

        pltpu.sync_copy(idx_hbm.at[pl.ds(base, CH)], idx_a)
        pltpu.async_copy(table_hbm.at[idx_a], rows_a, sem_a)

        def body(j, carry):
            ia = 2 * j
            ib = 2 * j + 1
            pltpu.sync_copy(idx_hbm.at[pl.ds(base + ib * CH, CH)], idx_b)
            pltpu.async_copy(table_hbm.at[idx_b], rows_b, sem_b)
            pltpu.make_async_copy(table_hbm.at[idx_a], rows_a, sem_a).wait()
            pltpu.sync_copy(rows_a, out_hbm.at[pl.ds(base + ia * CH, CH)])

            @pl.when(ib + 1 < nch)
            def _():
                pltpu.sync_copy(
                    idx_hbm.at[pl.ds(base + (ib + 1) * CH, CH)], idx_a)
                pltpu.async_copy(table_hbm.at[idx_a], rows_a, sem_a)

            pltpu.make_async_copy(table_hbm.at[idx_b], rows_b, sem_b).wait()
            pltpu.sync_copy(rows_b, out_hbm.at[pl.ds(base + ib * CH, CH)])
            return carry

        lax.fori_loop(0, nch // 2, body, 0)

    return gk(table, idx_flat)


def _make_pass_body(n_norm, n_mat, emit_max, C):
    def body(u_ref, v_ref, *refs):
        o_ref = refs[-1]
        prm = refs[:-1]
        t = pl.program_id(1)
        u = u_ref[0]
        if u.shape[1] != C:
            u = u[:, :C]
        h = u.reshape(PTS, KNB, C) - v_ref[0].reshape(PTS, 1, C)
        pi = 0
        for j in range(n_norm):
            nrm = prm[pi][0]
            pi += 1
            h = jnp.maximum(h * nrm[0:1].reshape(1, 1, C)
                            + nrm[1:2].reshape(1, 1, C), 0.0)
            if j < n_mat:
                w = prm[pi][...]
                bb = prm[pi + 1][...]
                pi += 2
                h = (jnp.dot(h.reshape(PTS * KNB, C), w,
                             preferred_element_type=jnp.float32, precision=lax.Precision.HIGHEST)
                     .reshape(PTS, KNB, C) + bb.reshape(1, 1, C))
        if emit_max:
            o_ref[0] = jnp.max(h, axis=1)
        else:
            s1 = jnp.sum(h, axis=(0, 1)).reshape(1, C)
            s2 = jnp.sum(h * h, axis=(0, 1)).reshape(1, C)
            upd = jnp.concatenate(
                [s1, s2, jnp.zeros((6, C), jnp.float32)], axis=0)

            @pl.when(t == 0)
            def _():
                o_ref[...] = jnp.zeros_like(o_ref)

            o_ref[...] += upd.reshape(1, 8, C)
    return body


def _stage_pass(U, v, norms, mats, emit_max):
    B, N, C = v.shape
    T = N // PTS
    in_specs = [
        pl.BlockSpec((1, PTS * KNB, U.shape[-1]), lambda b, t: (b, t, 0)),
        pl.BlockSpec((1, PTS, C), lambda b, t: (b, t, 0)),
    ]
    args = [U, v]
    for j in range(len(norms)):
        in_specs.append(pl.BlockSpec((1, 8, C), lambda b, t: (b, 0, 0)))
        args.append(norms[j])
        if j < len(mats):
            w, bb = mats[j]
            in_specs.append(pl.BlockSpec((C, C), lambda b, t: (0, 0)))
            in_specs.append(pl.BlockSpec((1, C), lambda b, t: (0, 0)))
            args.append(w)
            args.append(bb.reshape(1, C))
    if emit_max:
        out_spec = pl.BlockSpec((1, PTS, C), lambda b, t: (b, t, 0))
        out_shape = jax.ShapeDtypeStruct((B, N, C), jnp.float32)
    else:
        out_spec = pl.BlockSpec((1, 8, C), lambda b, t: (b, 0, 0))
        out_shape = jax.ShapeDtypeStruct((B, 8, C), jnp.float32)
    return pl.pallas_call(
        _make_pass_body(len(norms), len(mats), emit_max, C),
        grid=(B, T),
        in_specs=in_specs,
        out_specs=out_spec,
        out_shape=out_shape,
    )(*args)


def _mk_norm(stats, g, be, rows):
    mu = stats[:, 0, :] / rows
    ex2 = stats[:, 1, :] / rows
    var = ex2 - mu * mu
    scale = g[None, :] / jnp.sqrt(var + EPS)
    shift = be[None, :] - mu * scale
    B, C = mu.shape
    return jnp.concatenate(
        [scale[:, None, :], shift[:, None, :],
         jnp.zeros((B, 6, C), jnp.float32)], axis=1)


def _mlp_a_body(x_ref, w_ref, b_ref, h_ref, s_ref):
    first = jnp.logical_and(pl.program_id(0) == 0, pl.program_id(1) == 0)
    h = (jnp.dot(x_ref[0], w_ref[...], preferred_element_type=jnp.float32, precision=lax.Precision.HIGHEST)
         + b_ref[...])
    h_ref[0] = h
    s1 = jnp.sum(h, axis=0).reshape(1, -1)
    s2 = jnp.sum(h * h, axis=0).reshape(1, -1)
    upd = jnp.concatenate(
        [s1, s2, jnp.zeros((6, h.shape[1]), jnp.float32)], axis=0)

    @pl.when(first)
    def _():
        s_ref[...] = jnp.zeros_like(s_ref)

    s_ref[...] += upd


def _mlp_a(x, w, b):
    B, N, C = x.shape
    Co = w.shape[1]
    RT = 512
    return pl.pallas_call(
        _mlp_a_body,
        grid=(B, N // RT),
        in_specs=[
            pl.BlockSpec((1, RT, C), lambda b, t: (b, t, 0)),
            pl.BlockSpec((C, Co), lambda b, t: (0, 0)),
            pl.BlockSpec((1, Co), lambda b, t: (0, 0)),
        ],
        out_specs=[
            pl.BlockSpec((1, RT, Co), lambda b, t: (b, t, 0)),
            pl.BlockSpec((8, Co), lambda b, t: (0, 0)),
        ],
        out_shape=[
            jax.ShapeDtypeStruct((B, N, Co), jnp.float32),
            jax.ShapeDtypeStruct((8, Co), jnp.float32),
        ],
    )(x, w, b.reshape(1, Co))


def _mlp_b_body(h_ref, n_ref, w_ref, b_ref, o_ref):
    nrm = n_ref[...]
    h = jnp.maximum(h_ref[0] * nrm[0:1] + nrm[1:2], 0.0)
    o_ref[0] = (jnp.dot(h, w_ref[...], preferred_element_type=jnp.float32, precision=lax.Precision.HIGHEST)
                + b_ref[...])


def _mlp_b(h4, nrm, w, b):
    B, N, C = h4.shape
    Co = w.shape[1]
    RT = 512
    return pl.pallas_call(
        _mlp_b_body,
        grid=(B, N // RT),
        in_specs=[
            pl.BlockSpec((1, RT, C), lambda b, t: (b, t, 0)),
            pl.BlockSpec((8, C), lambda b, t: (0, 0)),
            pl.BlockSpec((C, Co), lambda b, t: (0, 0)),
            pl.BlockSpec((1, Co), lambda b, t: (0, 0)),
        ],
        out_specs=pl.BlockSpec((1, RT, Co), lambda b, t: (b, t, 0)),
        out_shape=jax.ShapeDtypeStruct((B, N, Co), jnp.float32),
    )(h4, nrm, w, b.reshape(1, Co))


def kernel(pc, params):
    B, N, _ = pc.shape
    rows = N * KNB

    sq = jnp.sum(pc * pc, axis=-1, keepdims=True)
    pcq = jnp.concatenate([pc, sq], axis=-1)
    pct = jnp.transpose(pcq, (0, 2, 1))
    pc8 = jnp.concatenate(
        [pc, jnp.zeros((B, N, 5), jnp.float32)], axis=-1)
    pc8t = jnp.transpose(pc8, (0, 2, 1))

    idx = _knn(pcq, pct, pc8, pc8t)
    idx_flat = idx.reshape(B * rows)

    x = pc
    for s in range(3):
        l0 = params["sc%d_0" % s]
        l1 = params["sc%d_1" % s]
        l2 = params["sc%d_2" % s]
        Cin = x.shape[-1]
        C = l0["w"].shape[1]
        wa = l0["w"][:Cin]
        wb8 = jnp.concatenate(
            [l0["w"][Cin:], jnp.zeros((5, C), jnp.float32)], axis=0)
        u, v = _compute_uv(x, pc8, wa, wb8, l0["b"])
        upad = u.reshape(B * N, C)
        if C < 128:
            upad = jnp.pad(upad, ((0, 0), (0, 128 - C)))
        U = _gather_rows(upad, idx_flat).reshape(B, rows, 128)
        if C < 128:
            U = U[:, :, :C]

        st1 = _stage_pass(U, v, [], [], False)
        n1 = _mk_norm(st1, l0["g"], l0["be"], rows)
        st2 = _stage_pass(U, v, [n1], [(l1["w"], l1["b"])], False)
        n2 = _mk_norm(st2, l1["g"], l1["be"], rows)
        st3 = _stage_pass(U, v, [n1, n2],
                          [(l1["w"], l1["b"]), (l2["w"], l2["b"])], False)
        n3 = _mk_norm(st3, l2["g"], l2["be"], rows)
        x = _stage_pass(U, v, [n1, n2, n3],
                        [(l1["w"], l1["b"]), (l2["w"], l2["b"])], True)

    h4, stm = _mlp_a(x, params["mlp0"]["w"], params["mlp0"]["b"])
    nm = _mk_norm(stm.reshape(1, 8, -1), params["mlp0"]["g"],
                  params["mlp0"]["be"], B * N)[0]
    out = _mlp_b(h4, nm, params["mlp1"]["w"], params["mlp1"]["b"])
    return jnp.transpose(out, (0, 2, 1))


# speedup vs baseline: 1.0387x; 1.0387x over previous
import functools

import jax
import jax.numpy as jnp
from jax import lax
from jax.experimental import pallas as pl
from jax.experimental.pallas import tpu as pltpu
from jax.experimental.pallas import tpu_sc as plsc

EPS = 1e-5
KNB = 32
PTS = 128
KNN_RT = 256


def _uv_body(x_ref, pc_ref, wa_ref, wb_ref, b_ref, u_ref, v_ref):
    x = x_ref[0]
    p = pc_ref[0]
    v = jnp.dot(p, wb_ref[...], preferred_element_type=jnp.float32, precision=lax.Precision.HIGHEST)
    u = (jnp.dot(x, wa_ref[...], preferred_element_type=jnp.float32, precision=lax.Precision.HIGHEST)
         + v + b_ref[...])
    u_ref[0] = u
    v_ref[0] = v


def _compute_uv(x, pc8, wa, wb, b1):
    B, N, Cin = x.shape
    C = wa.shape[1]
    return pl.pallas_call(
        _uv_body,
        grid=(B,),
        in_specs=[
            pl.BlockSpec((1, N, Cin), lambda b: (b, 0, 0)),
            pl.BlockSpec((1, N, 8), lambda b: (b, 0, 0)),
            pl.BlockSpec((Cin, C), lambda b: (0, 0)),
            pl.BlockSpec((8, C), lambda b: (0, 0)),
            pl.BlockSpec((1, C), lambda b: (0, 0)),
        ],
        out_specs=[
            pl.BlockSpec((1, N, C), lambda b: (b, 0, 0)),
            pl.BlockSpec((1, N, C), lambda b: (b, 0, 0)),
        ],
        out_shape=[
            jax.ShapeDtypeStruct((B, N, C), jnp.float32),
            jax.ShapeDtypeStruct((B, N, C), jnp.float32),
        ],
    )(x, pc8, wa, wb, b1.reshape(1, C))


def _knn_body(q_ref, pt_ref, q8_ref, p8t_ref, idx_ref):
    b = pl.program_id(0)
    q = q_ref[0]
    pt = pt_ref[0]
    n = pt.shape[1]
    rt = q.shape[0]
    e = jnp.dot(q8_ref[0], p8t_ref[0], preferred_element_type=jnp.float32,
                precision=lax.Precision.HIGHEST)
    d = (q[:, 3:4] + pt[3:4, :]) - 2.0 * e
    iota = lax.broadcasted_iota(jnp.int32, (rt, n), 1)
    base = b * n
    cols = []
    for _ in range(KNB):
        m = jnp.min(d, axis=1, keepdims=True)
        eq = d == m
        cols.append(jnp.min(jnp.where(eq, iota, n), axis=1) + base)
        d = jnp.where(eq, jnp.float32(jnp.inf), d)
    idx_ref[0] = jnp.stack(cols, axis=1)


def _knn(pcq, pct, pc8, pc8t):
    B, N, _ = pcq.shape
    return pl.pallas_call(
        _knn_body,
        grid=(B, N // KNN_RT),
        in_specs=[
            pl.BlockSpec((1, KNN_RT, 4), lambda b, t: (b, t, 0)),
            pl.BlockSpec((1, 4, N), lambda b, t: (b, 0, 0)),
            pl.BlockSpec((1, KNN_RT, 8), lambda b, t: (b, t, 0)),
            pl.BlockSpec((1, 8, N), lambda b, t: (b, 0, 0)),
        ],
        out_specs=pl.BlockSpec((1, KNN_RT, KNB), lambda b, t: (b, t, 0)),
        out_shape=jax.ShapeDtypeStruct((B, N, KNB), jnp.int32),
    )(pcq, pct, pc8, pc8t)


def _gather_rows(table, idx_flat):
    R, C = idx_flat.shape[0], table.shape[1]
    NW = 32
    CH = 128
    per = R // NW
    nch = per // CH
    mesh = plsc.VectorSubcoreMesh(core_axis_name="c", subcore_axis_name="s")

    @functools.partial(
        pl.kernel,
        mesh=mesh,
        out_type=jax.ShapeDtypeStruct((R, C), jnp.float32),
        scratch_types=[
            pltpu.VMEM((CH,), jnp.int32),
            pltpu.VMEM((CH,), jnp.int32),
            pltpu.VMEM((CH, C), jnp.float32),
            pltpu.VMEM((CH, C), jnp.float32),
            pltpu.SemaphoreType.DMA,
            pltpu.SemaphoreType.DMA,
        ],
    )
    def gk(table_hbm, idx_hbm, out_hbm, idx_a, idx_b, rows_a, rows_b,
           sem_a, sem_b):
        wid = lax.axis_index("s") * 2 + lax.axis_index("c")
        base = wid * per

        pltpu.sync_copy(idx_hbm.at[pl.ds(base, CH)], idx_a)
        pltpu.async_copy(table_hbm.at[idx_a], rows_a, sem_a)

        def body(j, carry):
            ia = 2 * j
            ib = 2 * j + 1
            pltpu.sync_copy(idx_hbm.at[pl.ds(base + ib * CH, CH)], idx_b)
            pltpu.async_copy(table_hbm.at[idx_b], rows_b, sem_b)
            pltpu.make_async_copy(table_hbm.at[idx_a], rows_a, sem_a).wait()
            pltpu.sync_copy(rows_a, out_hbm.at[pl.ds(base + ia * CH, CH)])

            @pl.when(ib + 1 < nch)
            def _():
                pltpu.sync_copy(
                    idx_hbm.at[pl.ds(base + (ib + 1) * CH, CH)], idx_a)
                pltpu.async_copy(table_hbm.at[idx_a], rows_a, sem_a)

            pltpu.make_async_copy(table_hbm.at[idx_b], rows_b, sem_b).wait()
            pltpu.sync_copy(rows_b, out_hbm.at[pl.ds(base + ib * CH, CH)])
            return carry

        lax.fori_loop(0, nch // 2, body, 0)

    return gk(table, idx_flat)


def _make_pass_body(n_norm, n_mat, emit_max, C):
    def body(u_ref, v_ref, *refs):
        o_ref = refs[-1]
        prm = refs[:-1]
        t = pl.program_id(1)
        u = u_ref[0]
        if u.shape[1] != C:
            u = u[:, :C]
        h = u.reshape(PTS, KNB, C) - v_ref[0].reshape(PTS, 1, C)
        pi = 0
        for j in range(n_norm):
            nrm = prm[pi][0]
            pi += 1
            h = jnp.maximum(h * nrm[0:1].reshape(1, 1, C)
                            + nrm[1:2].reshape(1, 1, C), 0.0)
            if j < n_mat:
                w = prm[pi][...]
                bb = prm[pi + 1][...]
                pi += 2
                h = (jnp.dot(h.reshape(PTS * KNB, C), w,
                             preferred_element_type=jnp.float32, precision=lax.Precision.HIGHEST)
                     .reshape(PTS, KNB, C) + bb.reshape(1, 1, C))
        if emit_max:
            o_ref[0] = jnp.max(h, axis=1)
        else:
            s1 = jnp.sum(h, axis=(0, 1)).reshape(1, C)
            s2 = jnp.sum(h * h, axis=(0, 1)).reshape(1, C)
            upd = jnp.concatenate(
                [s1, s2, jnp.zeros((6, C), jnp.float32)], axis=0)

            @pl.when(t == 0)
            def _():
                o_ref[...] = jnp.zeros_like(o_ref)

            o_ref[...] += upd.reshape(1, 8, C)
    return body


def _stage_pass(U, v, norms, mats, emit_max):
    B, N, C = v.shape
    T = N // PTS
    in_specs = [
        pl.BlockSpec((1, PTS * KNB, U.shape[-1]), lambda b, t: (b, t, 0)),
        pl.BlockSpec((1, PTS, C), lambda b, t: (b, t, 0)),
    ]
    args = [U, v]
    for j in range(len(norms)):
        in_specs.append(pl.BlockSpec((1, 8, C), lambda b, t: (b, 0, 0)))
        args.append(norms[j])
        if j < len(mats):
            w, bb = mats[j]
            in_specs.append(pl.BlockSpec((C, C), lambda b, t: (0, 0)))
            in_specs.append(pl.BlockSpec((1, C), lambda b, t: (0, 0)))
            args.append(w)
            args.append(bb.reshape(1, C))
    if emit_max:
        out_spec = pl.BlockSpec((1, PTS, C), lambda b, t: (b, t, 0))
        out_shape = jax.ShapeDtypeStruct((B, N, C), jnp.float32)
    else:
        out_spec = pl.BlockSpec((1, 8, C), lambda b, t: (b, 0, 0))
        out_shape = jax.ShapeDtypeStruct((B, 8, C), jnp.float32)
    return pl.pallas_call(
        _make_pass_body(len(norms), len(mats), emit_max, C),
        grid=(B, T),
        in_specs=in_specs,
        out_specs=out_spec,
        out_shape=out_shape,
    )(*args)


def _mk_norm(stats, g, be, rows):
    mu = stats[:, 0, :] / rows
    ex2 = stats[:, 1, :] / rows
    var = ex2 - mu * mu
    scale = g[None, :] / jnp.sqrt(var + EPS)
    shift = be[None, :] - mu * scale
    B, C = mu.shape
    return jnp.concatenate(
        [scale[:, None, :], shift[:, None, :],
         jnp.zeros((B, 6, C), jnp.float32)], axis=1)


def _mlp_a_body(x_ref, w_ref, b_ref, h_ref, s_ref):
    first = jnp.logical_and(pl.program_id(0) == 0, pl.program_id(1) == 0)
    h = (jnp.dot(x_ref[0], w_ref[...], preferred_element_type=jnp.float32, precision=lax.Precision.HIGHEST)
         + b_ref[...])
    h_ref[0] = h
    s1 = jnp.sum(h, axis=0).reshape(1, -1)
    s2 = jnp.sum(h * h, axis=0).reshape(1, -1)
    upd = jnp.concatenate(
        [s1, s2, jnp.zeros((6, h.shape[1]), jnp.float32)], axis=0)

    @pl.when(first)
    def _():
        s_ref[...] = jnp.zeros_like(s_ref)

    s_ref[...] += upd


def _mlp_a(x, w, b):
    B, N, C = x.shape
    Co = w.shape[1]
    RT = 512
    return pl.pallas_call(
        _mlp_a_body,
        grid=(B, N // RT),
        in_specs=[
            pl.BlockSpec((1, RT, C), lambda b, t: (b, t, 0)),
            pl.BlockSpec((C, Co), lambda b, t: (0, 0)),
            pl.BlockSpec((1, Co), lambda b, t: (0, 0)),
        ],
        out_specs=[
            pl.BlockSpec((1, RT, Co), lambda b, t: (b, t, 0)),
            pl.BlockSpec((8, Co), lambda b, t: (0, 0)),
        ],
        out_shape=[
            jax.ShapeDtypeStruct((B, N, Co), jnp.float32),
            jax.ShapeDtypeStruct((8, Co), jnp.float32),
        ],
    )(x, w, b.reshape(1, Co))


def _mlp_b_body(h_ref, n_ref, w_ref, b_ref, o_ref):
    nrm = n_ref[...]
    h = jnp.maximum(h_ref[0] * nrm[0:1] + nrm[1:2], 0.0)
    o_ref[0] = (jnp.dot(h, w_ref[...], preferred_element_type=jnp.float32, precision=lax.Precision.HIGHEST)
                + b_ref[...])


def _mlp_b(h4, nrm, w, b):
    B, N, C = h4.shape
    Co = w.shape[1]
    RT = 512
    return pl.pallas_call(
        _mlp_b_body,
        grid=(B, N // RT),
        in_specs=[
            pl.BlockSpec((1, RT, C), lambda b, t: (b, t, 0)),
            pl.BlockSpec((8, C), lambda b, t: (0, 0)),
            pl.BlockSpec((C, Co), lambda b, t: (0, 0)),
            pl.BlockSpec((1, Co), lambda b, t: (0, 0)),
        ],
        out_specs=pl.BlockSpec((1, RT, Co), lambda b, t: (b, t, 0)),
        out_shape=jax.ShapeDtypeStruct((B, N, Co), jnp.float32),
    )(h4, nrm, w, b.reshape(1, Co))


def kernel(pc, params):
    B, N, _ = pc.shape
    rows = N * KNB

    sq = jnp.sum(pc * pc, axis=-1, keepdims=True)
    pcq = jnp.concatenate([pc, sq], axis=-1)
    pct = jnp.transpose(pcq, (0, 2, 1))
    pc8 = jnp.concatenate(
        [pc, jnp.zeros((B, N, 5), jnp.float32)], axis=-1)
    pc8t = jnp.transpose(pc8, (0, 2, 1))

    idx = _knn(pcq, pct, pc8, pc8t)
    idx_flat = idx.reshape(B * rows)

    x = pc
    for s in range(3):
        l0 = params["sc%d_0" % s]
        l1 = params["sc%d_1" % s]
        l2 = params["sc%d_2" % s]
        Cin = x.shape[-1]
        C = l0["w"].shape[1]
        wa = l0["w"][:Cin]
        wb8 = jnp.concatenate(
            [l0["w"][Cin:], jnp.zeros((5, C), jnp.float32)], axis=0)
        u, v = _compute_uv(x, pc8, wa, wb8, l0["b"])
        upad = u.reshape(B * N, C)
        if C < 128:
            upad = jnp.pad(upad, ((0, 0), (0, 128 - C)))
        U = _gather_rows(upad, idx_flat).reshape(B, rows, 128)
        if C < 128:
            U = U[:, :, :C]

        st1 = _stage_pass(U, v, [], [], False)
        n1 = _mk_norm(st1, l0["g"], l0["be"], rows)
        st2 = _stage_pass(U, v, [n1], [(l1["w"], l1["b"])], False)
        n2 = _mk_norm(st2, l1["g"], l1["be"], rows)
        st3 = _stage_pass(U, v, [n1, n2],
                          [(l1["w"], l1["b"]), (l2["w"], l2["b"])], False)
        n3 = _mk_norm(st3, l2["g"], l2["be"], rows)
        x = _stage_pass(U, v, [n1, n2, n3],
                        [(l1["w"], l1["b"]), (l2["w"], l2["b"])], True)

    h4, stm = _mlp_a(x, params["mlp0"]["w"], params["mlp0"]["b"])
    nm = _mk_norm(stm.reshape(1, 8, -1), params["mlp0"]["g"],
                  params["mlp0"]["be"], B * N)[0]
    out = _mlp_b(h4, nm, params["mlp1"]["w"], params["mlp1"]["b"])
    return jnp.transpose(out, (0, 2, 1))
